# baseline (device time: 87053 ns/iter reference)
import jax
import jax.numpy as jnp
from jax import lax
from jax.experimental import pallas as pl
from jax.experimental.pallas import tpu as pltpu

N_DEV = 4
SQ = 2048
D_MODEL = 1024
HQ = 8
DH = 128
SCALE = 0.08838834764831843
BLK = 64
N_BLK = SQ // BLK
GROUPS = 4
BLK_PER_GROUP = N_BLK // GROUPS
CHUNK = SQ // N_DEV
HALFC = CHUNK // 2
GRP_ROWS = BLK * BLK_PER_GROUP
GC = GRP_ROWS // N_DEV
N_HOP = 2
N_TICK = GROUPS + N_HOP


def _body(x_ref, wq_ref, kext_ref, vext_ref, wo_ref, out_ref,
          k_vmem, v_vmem, grp_ref, rsb, agb,
          kv_sems, rs_send, rs_recv, ag_send, ag_recv):
    my = lax.axis_index("i")
    peers = [lax.rem(my + 1 + j, N_DEV) for j in range(3)]
    bf16 = jnp.bfloat16
    f32 = jnp.float32

    cp_k = pltpu.make_async_copy(
        kext_ref.at[0, :, pl.ds(my * HQ, HQ), :], k_vmem, kv_sems.at[0]
    )
    cp_v = pltpu.make_async_copy(
        vext_ref.at[0, :, pl.ds(my * HQ, HQ), :], v_vmem, kv_sems.at[1]
    )
    cp_k.start()
    cp_v.start()

    wq_b = wq_ref[:, :].astype(bf16)
    wo_b = wo_ref[:, :].astype(bf16)
    cp_k.wait()
    cp_v.wait()

    def compute_group(r):
        offs = [BLK * (GROUPS * m + r) for m in range(BLK_PER_GROUP)]
        Xr = jnp.concatenate(
            [x_ref[o:o + BLK, :] for o in offs], axis=0).astype(bf16)
        Kr = jnp.concatenate(
            [k_vmem[o:o + BLK, :, :] for o in offs], axis=0
        ).reshape(GRP_ROWS, HQ * DH).astype(bf16)
        Vr = jnp.concatenate(
            [v_vmem[o:o + BLK, :, :] for o in offs], axis=0
        ).reshape(GRP_ROWS, HQ * DH).astype(bf16)
        Qr = jnp.dot(Xr, wq_b, preferred_element_type=f32).astype(bf16)
        ctx_parts = []
        for h in range(HQ):
            cs = slice(DH * h, DH * (h + 1))
            s = lax.dot_general(
                Qr[:, cs], Kr[:, cs],
                dimension_numbers=(((1,), (1,)), ((), ())),
                preferred_element_type=f32,
            ) * SCALE
            e = jnp.exp(s)
            w = (e * (1.0 / jnp.sum(e, axis=-1, keepdims=True))).astype(bf16)
            ctx_parts.append(
                jnp.dot(w, Vr[:, cs], preferred_element_type=f32)
            )
        ctx_r = jnp.concatenate(ctx_parts, axis=1).astype(bf16)
        grp_ref[r, :, :] = jnp.dot(
            ctx_r, wo_b, preferred_element_type=f32).astype(bf16)

    def store_subchunk(r, c, rows_f32):
        out_ref[pl.ds(c * CHUNK + r * BLK, BLK), :] = rows_f32[0:BLK, :]
        out_ref[pl.ds(c * CHUNK + HALFC + r * BLK, BLK), :] = rows_f32[BLK:2 * BLK, :]

    descs = {}

    def issue_hop(r, s):
        ds = []
        for j in range(3):
            if s == 0:
                src = grp_ref.at[r, pl.ds(peers[j] * GC, GC), :]
                dst, ssem, rsem = rsb.at[r, j], rs_send.at[r, j], rs_recv.at[r, j]
            else:
                src = grp_ref.at[r, pl.ds(my * GC, GC), :]
                dst, ssem, rsem = agb.at[r, j], ag_send.at[r, j], ag_recv.at[r, j]
            d = pltpu.make_async_remote_copy(
                src_ref=src, dst_ref=dst, send_sem=ssem, recv_sem=rsem,
                device_id=(peers[j],), device_id_type=pl.DeviceIdType.MESH,
            )
            d.start()
            ds.append(d)
        descs[(r, s)] = ds

    def wait_hop(r, s):
        ds = descs[(r, s)]
        if s == 0:
            for d in ds:
                d.wait()
            mine = pl.ds(my * GC, GC)
            acc = (grp_ref[r, mine, :].astype(f32)
                   + rsb[r, 0].astype(f32)
                   + rsb[r, 1].astype(f32)
                   + rsb[r, 2].astype(f32))
            store_subchunk(r, my, acc)
            grp_ref[r, mine, :] = acc.astype(bf16)
        else:
            for j in range(3):
                ds[j].wait()
                sender = lax.rem(my + 2 * N_DEV - 1 - j, N_DEV)
                store_subchunk(r, sender, agb[r, j].astype(f32))

    for tick in range(N_TICK):
        if tick < GROUPS:
            compute_group(tick)
        if tick == 0:
            barrier_sem = pltpu.get_barrier_semaphore()
            for j in range(3):
                pl.semaphore_signal(
                    barrier_sem, inc=1,
                    device_id=(peers[j],), device_id_type=pl.DeviceIdType.MESH,
                )
            pl.semaphore_wait(barrier_sem, 3)
        for r in range(GROUPS):
            s = tick - r - 1
            if 0 <= s <= N_HOP - 1:
                wait_hop(r, s)
                if s < N_HOP - 1:
                    issue_hop(r, s + 1)
        if tick < GROUPS:
            issue_hop(tick, 0)


def kernel(x, Wq, K_ext, V_ext, Wo):
    x2 = x.reshape(SQ, D_MODEL)

    out = pl.pallas_call(
        _body,
        out_shape=jax.ShapeDtypeStruct((SQ, D_MODEL), jnp.float32),
        in_specs=[
            pl.BlockSpec(memory_space=pltpu.MemorySpace.VMEM),
            pl.BlockSpec(memory_space=pltpu.MemorySpace.VMEM),
            pl.BlockSpec(memory_space=pltpu.MemorySpace.HBM),
            pl.BlockSpec(memory_space=pltpu.MemorySpace.HBM),
            pl.BlockSpec(memory_space=pltpu.MemorySpace.VMEM),
        ],
        out_specs=pl.BlockSpec(memory_space=pltpu.MemorySpace.VMEM),
        scratch_shapes=[
            pltpu.VMEM((SQ, HQ, DH), jnp.float32),
            pltpu.VMEM((SQ, HQ, DH), jnp.float32),
            pltpu.VMEM((GROUPS, GRP_ROWS, D_MODEL), jnp.bfloat16),
            pltpu.VMEM((GROUPS, 3, GC, D_MODEL), jnp.bfloat16),
            pltpu.VMEM((GROUPS, 3, GC, D_MODEL), jnp.bfloat16),
            pltpu.SemaphoreType.DMA((2,)),
            pltpu.SemaphoreType.DMA((GROUPS, 3)),
            pltpu.SemaphoreType.DMA((GROUPS, 3)),
            pltpu.SemaphoreType.DMA((GROUPS, 3)),
            pltpu.SemaphoreType.DMA((GROUPS, 3)),
        ],
        compiler_params=pltpu.CompilerParams(
            collective_id=0,
            vmem_limit_bytes=100 * 1024 * 1024,
        ),
    )(x2, Wq, K_ext, V_ext, Wo)
    return out.reshape(1, SQ, D_MODEL)
